# Initial kernel scaffold; baseline (speedup 1.0000x reference)
import math

import jax
import jax.numpy as jnp
from jax import lax
from jax.experimental import pallas as pl
from jax.experimental.pallas import tpu as pltpu
from jax.experimental.pallas import tpu_sc as plsc

N = 10000
E = 160000
D = 128
DA = 16
EPS = 1e-09
NORM = 1.0 / math.sqrt(20.0)

TN = 400
TE = 256
C = 128
NCHUNKS = E // C
NW = 32
NS = 16
ROWS_PER_SUB = N // NS


def _rownorm(y):
    m = jnp.mean(y, axis=1, keepdims=True)
    c = y - m
    var = jnp.sum(c * c, axis=1, keepdims=True) * (1.0 / (D - 1))
    return y / (jnp.sqrt(var) + EPS)


def _silu(z):
    return z * jax.nn.sigmoid(z)


def _stage_a_body(xn_ref, attr_ref, wbt_ref, wlt_ref, bl_ref, out_ref):
    x1 = xn_ref[...]
    x2 = attr_ref[...]
    outer = jnp.concatenate([x2[:, j:j + 1] * x1 for j in range(DA)], axis=1)
    xbi = jnp.dot(outer, wbt_ref[...], preferred_element_type=jnp.float32)
    wlt = wlt_ref[...]
    y = (jnp.dot(x1, wlt[:D], preferred_element_type=jnp.float32)
         + jnp.dot(x2, wlt[D:D + DA], preferred_element_type=jnp.float32)
         + jnp.dot(xbi, wlt[D + DA:], preferred_element_type=jnp.float32)
         + bl_ref[...])
    out_ref[...] = _rownorm(y)


def _gather_body(table, srci, dsti, srows, drows, sidx, didx, sbuf, dbuf,
                 sem1, sem2):
    c = lax.axis_index("c")
    s = lax.axis_index("s")
    wid = s * 2 + c
    nchunks = jnp.where(wid < NCHUNKS % NW, NCHUNKS // NW + 1, NCHUNKS // NW)

    def body(k, carry):
        base = (wid + NW * k) * C
        pltpu.sync_copy(srci.at[pl.ds(base, C)], sidx)
        pltpu.sync_copy(dsti.at[pl.ds(base, C)], didx)
        cp1 = pltpu.async_copy(table.at[sidx], sbuf, sem1)
        cp2 = pltpu.async_copy(table.at[didx], dbuf, sem2)
        cp1.wait()
        cp2.wait()
        pltpu.sync_copy(sbuf, srows.at[pl.ds(base, C)])
        pltpu.sync_copy(dbuf, drows.at[pl.ds(base, C)])
        return carry

    lax.fori_loop(0, nchunks, body, 0)


def _stage_c_body(src_ref, dst_ref, a_ref, fc1_ref, fc2_ref, wbt_ref,
                  wlt1_ref, bl1_ref, wbxe_ref, wlt2_ref, bl2_ref, out_ref):
    s = src_ref[...]
    d = dst_ref[...]
    a = a_ref[...]
    w = _silu(a * fc1_ref[...])
    grad = w * (s - d)
    ave = w * (s + d) * 0.5
    xbi = jnp.zeros((TE, D), jnp.float32)
    for j0 in range(0, D, 32):
        outer = jnp.concatenate(
            [ave[:, j:j + 1] * grad for j in range(j0, j0 + 32)], axis=1)
        xbi = xbi + jnp.dot(outer, wbt_ref[pl.ds(j0 * D, 32 * D), :],
                            preferred_element_type=jnp.float32)
    wlt = wlt1_ref[...]
    xe = (jnp.dot(grad, wlt[:D], preferred_element_type=jnp.float32)
          + jnp.dot(ave, wlt[D:2 * D], preferred_element_type=jnp.float32)
          + jnp.dot(xbi, wlt[2 * D:], preferred_element_type=jnp.float32)
          + bl1_ref[...])
    xbi2 = jnp.dot(xe, wbxe_ref[...], preferred_element_type=jnp.float32) * a
    wlt2 = wlt2_ref[...]
    xe2 = (jnp.dot(xe, wlt2[:D], preferred_element_type=jnp.float32)
           + a * wlt2[D:D + 1]
           + jnp.dot(xbi2, wlt2[D + 1:], preferred_element_type=jnp.float32)
           + bl2_ref[...])
    xe2 = _rownorm(xe2)
    w2 = _silu(a * fc2_ref[...])
    out_ref[...] = w2 * xe2 * NORM


def _scatter_body(vals, dsti, srci, zrows, xn1, xn2, idxb, vbuf, acc):
    c = lax.axis_index("c")
    s = lax.axis_index("s")
    pltpu.sync_copy(zrows.at[pl.ds(s * ROWS_PER_SUB, ROWS_PER_SUB)],
                    acc.at[pl.ds(s * ROWS_PER_SUB, ROWS_PER_SUB)])
    plsc.subcore_barrier()

    def run(idx_hbm):
        n = jnp.where(s < NCHUNKS % NS, NCHUNKS // NS + 1, NCHUNKS // NS)

        def body(k, carry):
            base = (s + NS * k) * C
            pltpu.sync_copy(idx_hbm.at[pl.ds(base, C)], idxb)
            pltpu.sync_copy(vals.at[pl.ds(base, C)], vbuf)
            pltpu.sync_copy(vbuf, acc.at[idxb], add=True)
            return carry

        lax.fori_loop(0, n, body, 0)

    @pl.when(c == 0)
    def _dst():
        run(dsti)

    @pl.when(c == 1)
    def _src():
        run(srci)

    plsc.subcore_barrier()

    @pl.when(c == 0)
    def _out1():
        pltpu.sync_copy(acc.at[pl.ds(s * ROWS_PER_SUB, ROWS_PER_SUB)],
                        xn1.at[pl.ds(s * ROWS_PER_SUB, ROWS_PER_SUB)])

    @pl.when(c == 1)
    def _out2():
        pltpu.sync_copy(acc.at[pl.ds(s * ROWS_PER_SUB, ROWS_PER_SUB)],
                        xn2.at[pl.ds(s * ROWS_PER_SUB, ROWS_PER_SUB)])


def _stage_e_body(x1_ref, x2_ref, wbt_ref, wlt_ref, bl_ref, out_ref):
    xn1 = x1_ref[...]
    xn2 = x2_ref[...]
    dd = xn1 - xn2
    sm = xn1 + xn2
    xbi = jnp.zeros((TN, D), jnp.float32)
    for j0 in range(0, D, 32):
        outer = jnp.concatenate(
            [sm[:, j:j + 1] * dd for j in range(j0, j0 + 32)], axis=1)
        xbi = xbi + jnp.dot(outer, wbt_ref[pl.ds(j0 * D, 32 * D), :],
                            preferred_element_type=jnp.float32)
    wlt = wlt_ref[...]
    y = (jnp.dot(dd, wlt[:D], preferred_element_type=jnp.float32)
         + jnp.dot(sm, wlt[D:2 * D], preferred_element_type=jnp.float32)
         + jnp.dot(xbi, wlt[2 * D:], preferred_element_type=jnp.float32)
         + bl_ref[...])
    out_ref[...] = _rownorm(_silu(y))


def kernel(xn, xn_attr, xe_attr, xe_src, xe_dst, Wb_xn, Wl_xn, bl_xn,
           W_fc1, b_fc1, Wb_n2e, Wl_n2e, bl_n2e, Wb_xe, Wl_xe, bl_xe,
           W_fc2, b_fc2, Wb_e2n, Wl_e2n, bl_e2n):
    f32 = jnp.float32
    xe_src = xe_src.astype(jnp.int32)
    xe_dst = xe_dst.astype(jnp.int32)

    wbt_xn = jnp.transpose(Wb_xn, (2, 1, 0)).reshape(DA * D, D)
    wbt_n2e = jnp.transpose(Wb_n2e, (2, 1, 0)).reshape(D * D, D)
    wbt_e2n = jnp.transpose(Wb_e2n, (2, 1, 0)).reshape(D * D, D)
    wlt_xn = Wl_xn.T
    wlt_n2e = Wl_n2e.T
    wlt_xe = Wl_xe.T
    wlt_e2n = Wl_e2n.T
    wbxe0 = Wb_xe[:, :, 0].T
    fc1 = W_fc1.T.reshape(1, D)
    fc2 = W_fc2.T.reshape(1, D)
    bl_xn2 = bl_xn.reshape(1, D)
    bl_n2e2 = bl_n2e.reshape(1, D)
    bl_xe2 = bl_xe.reshape(1, D)
    bl_e2n2 = bl_e2n.reshape(1, D)
    fc1b = b_fc1.reshape(1, D)
    fc2b = b_fc2.reshape(1, D)

    xnm = pl.pallas_call(
        _stage_a_body,
        grid=(N // TN,),
        in_specs=[
            pl.BlockSpec((TN, D), lambda i: (i, 0)),
            pl.BlockSpec((TN, DA), lambda i: (i, 0)),
            pl.BlockSpec((DA * D, D), lambda i: (0, 0)),
            pl.BlockSpec((2 * D + DA, D), lambda i: (0, 0)),
            pl.BlockSpec((1, D), lambda i: (0, 0)),
        ],
        out_specs=pl.BlockSpec((TN, D), lambda i: (i, 0)),
        out_shape=jax.ShapeDtypeStruct((N, D), f32),
    )(xn, xn_attr, wbt_xn, wlt_xn, bl_xn2)

    mesh = plsc.VectorSubcoreMesh(core_axis_name="c", subcore_axis_name="s")
    srows, drows = pl.kernel(
        _gather_body,
        out_type=[jax.ShapeDtypeStruct((E, D), f32),
                  jax.ShapeDtypeStruct((E, D), f32)],
        mesh=mesh,
        scratch_types=[
            pltpu.VMEM((C,), jnp.int32),
            pltpu.VMEM((C,), jnp.int32),
            pltpu.VMEM((C, D), f32),
            pltpu.VMEM((C, D), f32),
            pltpu.SemaphoreType.DMA,
            pltpu.SemaphoreType.DMA,
        ],
    )(xnm, xe_src, xe_dst)

    vals = pl.pallas_call(
        _stage_c_body,
        grid=(E // TE,),
        in_specs=[
            pl.BlockSpec((TE, D), lambda i: (i, 0)),
            pl.BlockSpec((TE, D), lambda i: (i, 0)),
            pl.BlockSpec((TE, 1), lambda i: (i, 0)),
            pl.BlockSpec((1, D), lambda i: (0, 0)),
            pl.BlockSpec((1, D), lambda i: (0, 0)),
            pl.BlockSpec((D * D, D), lambda i: (0, 0)),
            pl.BlockSpec((3 * D, D), lambda i: (0, 0)),
            pl.BlockSpec((1, D), lambda i: (0, 0)),
            pl.BlockSpec((D, D), lambda i: (0, 0)),
            pl.BlockSpec((2 * D + 1, D), lambda i: (0, 0)),
            pl.BlockSpec((1, D), lambda i: (0, 0)),
        ],
        out_specs=pl.BlockSpec((TE, D), lambda i: (i, 0)),
        out_shape=jax.ShapeDtypeStruct((E, D), f32),
    )(srows, drows, xe_attr, fc1, fc2, wbt_n2e, wlt_n2e, bl_n2e2,
      wbxe0, wlt_xe, bl_xe2)
    del fc1b, fc2b

    zrows = jnp.zeros((N, D), f32)
    xn1, xn2 = pl.kernel(
        _scatter_body,
        out_type=[jax.ShapeDtypeStruct((N, D), f32),
                  jax.ShapeDtypeStruct((N, D), f32)],
        mesh=plsc.VectorSubcoreMesh(core_axis_name="c", subcore_axis_name="s"),
        scratch_types=[
            pltpu.VMEM((C,), jnp.int32),
            pltpu.VMEM((C, D), f32),
            pltpu.VMEM_SHARED((N, D), f32),
        ],
    )(vals, xe_dst, xe_src, zrows)

    out = pl.pallas_call(
        _stage_e_body,
        grid=(N // TN,),
        in_specs=[
            pl.BlockSpec((TN, D), lambda i: (i, 0)),
            pl.BlockSpec((TN, D), lambda i: (i, 0)),
            pl.BlockSpec((D * D, D), lambda i: (0, 0)),
            pl.BlockSpec((3 * D, D), lambda i: (0, 0)),
            pl.BlockSpec((1, D), lambda i: (0, 0)),
        ],
        out_specs=pl.BlockSpec((TN, D), lambda i: (i, 0)),
        out_shape=jax.ShapeDtypeStruct((N, D), f32),
    )(xn1, xn2, wbt_e2n, wlt_e2n, bl_e2n2)
    return out


# speedup vs baseline: 2.7340x; 2.7340x over previous
import math

import jax
import jax.numpy as jnp
from jax import lax
from jax.experimental import pallas as pl
from jax.experimental.pallas import tpu as pltpu
from jax.experimental.pallas import tpu_sc as plsc

N = 10000
E = 160000
D = 128
DA = 16
EPS = 1e-09
NORM = 1.0 / math.sqrt(20.0)

TN = 400
TE = 256
C = 128
NCHUNKS = E // C
NW = 32
NS = 16
ROWS_PER_SUB = 624
ROWS_TAIL = N - NS * ROWS_PER_SUB


def _rownorm(y):
    m = jnp.mean(y, axis=1, keepdims=True)
    c = y - m
    var = jnp.sum(c * c, axis=1, keepdims=True) * (1.0 / (D - 1))
    return y / (jnp.sqrt(var) + EPS)


def _silu(z):
    return z * jax.nn.sigmoid(z)


def _stage_a_body(xn_ref, attr_ref, wbt_ref, wlt_ref, bl_ref, out_ref):
    x1 = xn_ref[...]
    x2 = attr_ref[...]
    outer = jnp.concatenate([x2[:, j:j + 1] * x1 for j in range(DA)], axis=1)
    xbi = jnp.dot(outer, wbt_ref[...], preferred_element_type=jnp.float32)
    wlt = wlt_ref[...]
    y = (jnp.dot(x1, wlt[:D], preferred_element_type=jnp.float32)
         + jnp.dot(x2, wlt[D:D + DA], preferred_element_type=jnp.float32)
         + jnp.dot(xbi, wlt[D + DA:], preferred_element_type=jnp.float32)
         + bl_ref[...])
    out_ref[...] = _rownorm(y)


def _gather_body(table, srci, dsti, srows, drows, sidx, didx, sbuf, dbuf,
                 sem1, sem2):
    c = lax.axis_index("c")
    s = lax.axis_index("s")
    wid = s * 2 + c
    nchunks = jnp.where(wid < NCHUNKS % NW, NCHUNKS // NW + 1, NCHUNKS // NW)

    def body(k, carry):
        base = (wid + NW * k) * C
        pltpu.sync_copy(srci.at[pl.ds(base, C)], sidx)
        pltpu.sync_copy(dsti.at[pl.ds(base, C)], didx)
        cp1 = pltpu.async_copy(table.at[sidx], sbuf, sem1)
        cp2 = pltpu.async_copy(table.at[didx], dbuf, sem2)
        cp1.wait()
        cp2.wait()
        pltpu.sync_copy(sbuf, srows.at[pl.ds(base, C)])
        pltpu.sync_copy(dbuf, drows.at[pl.ds(base, C)])
        return carry

    lax.fori_loop(0, nchunks, body, 0)


def _stage_c_body(src_ref, dst_ref, a_ref, fc1_ref, bfc1_ref, fc2_ref,
                  bfc2_ref, wbt_ref, wlt1_ref, bl1_ref, wbxe_ref, wlt2_ref,
                  bl2_ref, out_ref):
    s = src_ref[...]
    d = dst_ref[...]
    a = a_ref[...]
    w = _silu(a * fc1_ref[...] + bfc1_ref[...])
    grad = w * (s - d)
    ave = w * (s + d) * 0.5
    xbi = jnp.zeros((TE, D), jnp.float32)
    for j0 in range(0, D, 32):
        outer = jnp.concatenate(
            [ave[:, j:j + 1] * grad for j in range(j0, j0 + 32)], axis=1)
        xbi = xbi + jnp.dot(outer, wbt_ref[pl.ds(j0 * D, 32 * D), :],
                            preferred_element_type=jnp.float32)
    wlt = wlt1_ref[...]
    xe = (jnp.dot(grad, wlt[:D], preferred_element_type=jnp.float32)
          + jnp.dot(ave, wlt[D:2 * D], preferred_element_type=jnp.float32)
          + jnp.dot(xbi, wlt[2 * D:], preferred_element_type=jnp.float32)
          + bl1_ref[...])
    xbi2 = jnp.dot(xe, wbxe_ref[...], preferred_element_type=jnp.float32) * a
    wlt2 = wlt2_ref[...]
    xe2 = (jnp.dot(xe, wlt2[:D], preferred_element_type=jnp.float32)
           + a * wlt2[D:D + 1]
           + jnp.dot(xbi2, wlt2[D + 1:], preferred_element_type=jnp.float32)
           + bl2_ref[...])
    xe2 = _rownorm(xe2)
    w2 = _silu(a * fc2_ref[...] + bfc2_ref[...])
    out_ref[...] = w2 * xe2 * NORM


def _scatter_body(vals, dsti, srci, zrows, xn1, xn2, idxb, vbuf, acc):
    c = lax.axis_index("c")
    s = lax.axis_index("s")
    pltpu.sync_copy(zrows.at[pl.ds(s * ROWS_PER_SUB, ROWS_PER_SUB)],
                    acc.at[pl.ds(s * ROWS_PER_SUB, ROWS_PER_SUB)])

    @pl.when(s == 0)
    def _ztail():
        pltpu.sync_copy(zrows.at[pl.ds(NS * ROWS_PER_SUB, ROWS_TAIL)],
                        acc.at[pl.ds(NS * ROWS_PER_SUB, ROWS_TAIL)])

    plsc.subcore_barrier()

    def run(idx_hbm):
        n = jnp.where(s < NCHUNKS % NS, NCHUNKS // NS + 1, NCHUNKS // NS)

        def body(k, carry):
            base = (s + NS * k) * C
            pltpu.sync_copy(idx_hbm.at[pl.ds(base, C)], idxb)
            pltpu.sync_copy(vals.at[pl.ds(base, C)], vbuf)
            pltpu.sync_copy(vbuf, acc.at[idxb], add=True)
            return carry

        lax.fori_loop(0, n, body, 0)

    @pl.when(c == 0)
    def _dst():
        run(dsti)

    @pl.when(c == 1)
    def _src():
        run(srci)

    plsc.subcore_barrier()

    @pl.when(c == 0)
    def _out1():
        pltpu.sync_copy(acc.at[pl.ds(s * ROWS_PER_SUB, ROWS_PER_SUB)],
                        xn1.at[pl.ds(s * ROWS_PER_SUB, ROWS_PER_SUB)])

        @pl.when(s == 0)
        def _t1():
            pltpu.sync_copy(acc.at[pl.ds(NS * ROWS_PER_SUB, ROWS_TAIL)],
                            xn1.at[pl.ds(NS * ROWS_PER_SUB, ROWS_TAIL)])

    @pl.when(c == 1)
    def _out2():
        pltpu.sync_copy(acc.at[pl.ds(s * ROWS_PER_SUB, ROWS_PER_SUB)],
                        xn2.at[pl.ds(s * ROWS_PER_SUB, ROWS_PER_SUB)])

        @pl.when(s == 0)
        def _t2():
            pltpu.sync_copy(acc.at[pl.ds(NS * ROWS_PER_SUB, ROWS_TAIL)],
                            xn2.at[pl.ds(NS * ROWS_PER_SUB, ROWS_TAIL)])


def _stage_e_body(x1_ref, x2_ref, wbt_ref, wlt_ref, bl_ref, out_ref):
    xn1 = x1_ref[...]
    xn2 = x2_ref[...]
    dd = xn1 - xn2
    sm = xn1 + xn2
    xbi = jnp.zeros((TN, D), jnp.float32)
    for j0 in range(0, D, 32):
        outer = jnp.concatenate(
            [sm[:, j:j + 1] * dd for j in range(j0, j0 + 32)], axis=1)
        xbi = xbi + jnp.dot(outer, wbt_ref[pl.ds(j0 * D, 32 * D), :],
                            preferred_element_type=jnp.float32)
    wlt = wlt_ref[...]
    y = (jnp.dot(dd, wlt[:D], preferred_element_type=jnp.float32)
         + jnp.dot(sm, wlt[D:2 * D], preferred_element_type=jnp.float32)
         + jnp.dot(xbi, wlt[2 * D:], preferred_element_type=jnp.float32)
         + bl_ref[...])
    out_ref[...] = _rownorm(_silu(y))


def kernel(xn, xn_attr, xe_attr, xe_src, xe_dst, Wb_xn, Wl_xn, bl_xn,
           W_fc1, b_fc1, Wb_n2e, Wl_n2e, bl_n2e, Wb_xe, Wl_xe, bl_xe,
           W_fc2, b_fc2, Wb_e2n, Wl_e2n, bl_e2n):
    f32 = jnp.float32
    xe_src = xe_src.astype(jnp.int32)
    xe_dst = xe_dst.astype(jnp.int32)

    wbt_xn = jnp.transpose(Wb_xn, (2, 1, 0)).reshape(DA * D, D)
    wbt_n2e = jnp.transpose(Wb_n2e, (2, 1, 0)).reshape(D * D, D)
    wbt_e2n = jnp.transpose(Wb_e2n, (2, 1, 0)).reshape(D * D, D)
    wlt_xn = Wl_xn.T
    wlt_n2e = Wl_n2e.T
    wlt_xe = Wl_xe.T
    wlt_e2n = Wl_e2n.T
    wbxe0 = Wb_xe[:, :, 0].T
    fc1 = W_fc1.T.reshape(1, D)
    fc2 = W_fc2.T.reshape(1, D)
    bl_xn2 = bl_xn.reshape(1, D)
    bl_n2e2 = bl_n2e.reshape(1, D)
    bl_xe2 = bl_xe.reshape(1, D)
    bl_e2n2 = bl_e2n.reshape(1, D)
    bfc1 = b_fc1.reshape(1, D)
    bfc2 = b_fc2.reshape(1, D)

    xnm = pl.pallas_call(
        _stage_a_body,
        grid=(N // TN,),
        in_specs=[
            pl.BlockSpec((TN, D), lambda i: (i, 0)),
            pl.BlockSpec((TN, DA), lambda i: (i, 0)),
            pl.BlockSpec((DA * D, D), lambda i: (0, 0)),
            pl.BlockSpec((2 * D + DA, D), lambda i: (0, 0)),
            pl.BlockSpec((1, D), lambda i: (0, 0)),
        ],
        out_specs=pl.BlockSpec((TN, D), lambda i: (i, 0)),
        out_shape=jax.ShapeDtypeStruct((N, D), f32),
    )(xn, xn_attr, wbt_xn, wlt_xn, bl_xn2)

    mesh = plsc.VectorSubcoreMesh(core_axis_name="c", subcore_axis_name="s")
    srows, drows = pl.kernel(
        _gather_body,
        out_type=[jax.ShapeDtypeStruct((E, D), f32),
                  jax.ShapeDtypeStruct((E, D), f32)],
        mesh=mesh,
        scratch_types=[
            pltpu.VMEM((C,), jnp.int32),
            pltpu.VMEM((C,), jnp.int32),
            pltpu.VMEM((C, D), f32),
            pltpu.VMEM((C, D), f32),
            pltpu.SemaphoreType.DMA,
            pltpu.SemaphoreType.DMA,
        ],
    )(xnm, xe_src, xe_dst)

    vals = pl.pallas_call(
        _stage_c_body,
        grid=(E // TE,),
        in_specs=[
            pl.BlockSpec((TE, D), lambda i: (i, 0)),
            pl.BlockSpec((TE, D), lambda i: (i, 0)),
            pl.BlockSpec((TE, 1), lambda i: (i, 0)),
            pl.BlockSpec((1, D), lambda i: (0, 0)),
            pl.BlockSpec((1, D), lambda i: (0, 0)),
            pl.BlockSpec((1, D), lambda i: (0, 0)),
            pl.BlockSpec((1, D), lambda i: (0, 0)),
            pl.BlockSpec((D * D, D), lambda i: (0, 0)),
            pl.BlockSpec((3 * D, D), lambda i: (0, 0)),
            pl.BlockSpec((1, D), lambda i: (0, 0)),
            pl.BlockSpec((D, D), lambda i: (0, 0)),
            pl.BlockSpec((2 * D + 1, D), lambda i: (0, 0)),
            pl.BlockSpec((1, D), lambda i: (0, 0)),
        ],
        out_specs=pl.BlockSpec((TE, D), lambda i: (i, 0)),
        out_shape=jax.ShapeDtypeStruct((E, D), f32),
    )(srows, drows, xe_attr, fc1, bfc1, fc2, bfc2, wbt_n2e, wlt_n2e, bl_n2e2,
      wbxe0, wlt_xe, bl_xe2)

    zrows = jnp.zeros((N, D), f32)
    xn1, xn2 = pl.kernel(
        _scatter_body,
        out_type=[jax.ShapeDtypeStruct((N, D), f32),
                  jax.ShapeDtypeStruct((N, D), f32)],
        mesh=plsc.VectorSubcoreMesh(core_axis_name="c", subcore_axis_name="s"),
        scratch_types=[
            pltpu.VMEM((C,), jnp.int32),
            pltpu.VMEM((C, D), f32),
            pltpu.VMEM_SHARED((N, D), f32),
        ],
    )(vals, xe_dst, xe_src, zrows)

    out = pl.pallas_call(
        _stage_e_body,
        grid=(N // TN,),
        in_specs=[
            pl.BlockSpec((TN, D), lambda i: (i, 0)),
            pl.BlockSpec((TN, D), lambda i: (i, 0)),
            pl.BlockSpec((D * D, D), lambda i: (0, 0)),
            pl.BlockSpec((3 * D, D), lambda i: (0, 0)),
            pl.BlockSpec((1, D), lambda i: (0, 0)),
        ],
        out_specs=pl.BlockSpec((TN, D), lambda i: (i, 0)),
        out_shape=jax.ShapeDtypeStruct((N, D), f32),
    )(xn1, xn2, wbt_e2n, wlt_e2n, bl_e2n2)
    return out
